# Initial kernel scaffold; baseline (speedup 1.0000x reference)
import jax
import jax.numpy as jnp
from jax.experimental import pallas as pl


def kernel(batch_size, agent_encodings, encode_coordinates):
    raise NotImplementedError("write your pallas kernel here")


# speedup vs baseline: 1.2889x; 1.2889x over previous
import functools

import jax
import jax.numpy as jnp
from jax import lax
from jax.experimental import pallas as pl
from jax.experimental.pallas import tpu as pltpu
from jax.experimental.pallas import tpu_sc as plsc

N = 131072
C = 64
NBATCH = 1024
SPA = 900
NC, NS = 2, 16
NW = NC * NS
APW = N // NW
HALF = N // NC
PAD = 192
STARTS_W = 1040
CH = 128
TILE = C * SPA
BPW = NBATCH // NW

_mesh = plsc.VectorSubcoreMesh(core_axis_name="c", subcore_axis_name="s")


@functools.partial(
    pl.kernel,
    out_type=(
        jax.ShapeDtypeStruct((N + PAD,), jnp.int32),
        jax.ShapeDtypeStruct((NC * STARTS_W,), jnp.int32),
    ),
    mesh=_mesh,
    scratch_types=[
        pltpu.VMEM((APW,), jnp.int32),
        pltpu.VMEM((APW,), jnp.int32),
        pltpu.VMEM((NBATCH * 16,), jnp.int32),
        pltpu.VMEM((NBATCH * 16,), jnp.int32),
        pltpu.VMEM((APW // CH, CH), jnp.int32),
        pltpu.VMEM((APW // CH, CH), jnp.int32),
        pltpu.VMEM((STARTS_W,), jnp.int32),
        pltpu.VMEM_SHARED((NS * NBATCH * 16,), jnp.int32),
        pltpu.SemaphoreType.DMA,
    ],
    compiler_params=pltpu.CompilerParams(needs_layout_passes=False, use_tc_tiling_on_sc=False),
)
def _bin_kernel(coords_hbm, sorted_hbm, starts_hbm,
                coords_v, barr, hist, stage, vals, dsts, starts_v,
                shared, sem):
    c = lax.axis_index("c")
    s = lax.axis_index("s")
    lane = lax.iota(jnp.int32, 16)
    zero16 = jnp.zeros((16,), jnp.int32)
    ones16 = jnp.ones((16,), jnp.int32)
    base = c * HALF + s * APW

    pltpu.sync_copy(coords_hbm.at[pl.ds(base, APW)], coords_v)

    def zero_body(i, _):
        hist[pl.ds(i * 16, 16)] = zero16
        return 0
    lax.fori_loop(0, NBATCH, zero_body, 0)

    def bin_body(t, _):
        cv = coords_v[pl.ds(t * 16, 16)]
        b = cv // SPA
        sloc = cv - b * SPA
        barr[pl.ds(t * 16, 16)] = b
        r = t // 8
        q = t - r * 8
        vals[r, pl.ds(q * 16, 16)] = ((base + t * 16 + lane) << 10) + sloc
        plsc.addupdate_scatter(hist, [b * 16 + lane], ones16)
        return 0
    lax.fori_loop(0, APW // 16, bin_body, 0)

    pltpu.sync_copy(hist, shared.at[pl.ds(s * NBATCH * 16, NBATCH * 16)])
    plsc.subcore_barrier()

    def chunk_body(k, sc_start):
        for w2 in range(NS):
            pltpu.sync_copy(shared.at[pl.ds(w2 * NBATCH * 16 + k * 1024, 1024)],
                            stage.at[pl.ds(w2 * 1024, 1024)])

        def bkt_body(b2, carry):
            start, sb_vec = carry
            total_vec = zero16
            below_vec = zero16
            own = zero16
            for w2 in range(NS):
                hv = stage[pl.ds(w2 * 1024 + b2 * 16, 16)]
                total_vec = total_vec + hv
                below_vec = below_vec + jnp.where(w2 < s, hv, zero16)
                own = jnp.where(w2 == s, hv, own)
            total = jnp.sum(total_vec)
            below = jnp.sum(below_vec)
            ex = plsc.cumsum(own) - own
            boff = k * 64 + b2
            hist[pl.ds(boff * 16, 16)] = start + below + ex
            sb_vec = jnp.where(lane == (b2 % 16), start, sb_vec)

            @pl.when(b2 % 16 == 15)
            def _():
                starts_v[pl.ds((boff // 16) * 16, 16)] = sb_vec
            return (start + total, sb_vec)

        out = lax.fori_loop(0, 64, bkt_body, (sc_start, zero16))
        return out[0]

    sc_total = lax.fori_loop(0, NBATCH // 64, chunk_body, jnp.int32(0))
    starts_v[pl.ds(NBATCH, 16)] = jnp.where(lane == 0, sc_total, zero16)

    @pl.when(s == 0)
    def _():
        pltpu.sync_copy(starts_v, starts_hbm.at[pl.ds(c * STARTS_W, STARTS_W)])

    def perm_body(t, _):
        b = barr[pl.ds(t * 16, 16)]
        idx = b * 16 + lane
        dst = plsc.load_gather(hist, [idx])
        plsc.store_scatter(hist, [idx], dst + 1)
        r = t // 8
        q = t - r * 8
        dsts[r, pl.ds(q * 16, 16)] = dst + c * HALF
        return 0
    lax.fori_loop(0, APW // 16, perm_body, 0)

    def dma_body(j, _):
        pltpu.async_copy(vals.at[j], sorted_hbm.at[dsts.at[j]], sem).wait()
        return 0
    lax.fori_loop(0, APW // CH, dma_body, 0)


@functools.partial(
    pl.kernel,
    out_type=jax.ShapeDtypeStruct((NBATCH * TILE,), jnp.float32),
    mesh=_mesh,
    scratch_types=[
        pltpu.VMEM((TILE,), jnp.float32),
        pltpu.VMEM((CH,), jnp.int32),
        pltpu.VMEM((CH,), jnp.int32),
        pltpu.VMEM((CH, C), jnp.float32),
        pltpu.VMEM((16,), jnp.int32),
        pltpu.SemaphoreType.DMA,
    ],
    compiler_params=pltpu.CompilerParams(needs_layout_passes=False, use_tc_tiling_on_sc=False),
)
def _acc_kernel(enc_hbm, sorted_hbm, starts_hbm, out_hbm,
                tile, ent_v, ids_v, rows_v, st_v, sem):
    c = lax.axis_index("c")
    s = lax.axis_index("s")
    lane = lax.iota(jnp.int32, 16)
    lane9 = lane * SPA
    wg = c * NS + s
    zero16f = jnp.zeros((16,), jnp.float32)
    ninf16 = jnp.full((16,), -jnp.inf, jnp.float32)

    def z_body(i, _):
        tile[pl.ds(i * 16, 16)] = zero16f
        return 0
    lax.fori_loop(0, TILE // 16, z_body, 0)

    def bucket_body(j, _):
        b = wg * BPW + j

        def seg(core):
            boff = (b // 8) * 8
            pltpu.sync_copy(starts_hbm.at[pl.ds(core * STARTS_W + boff, 16)], st_v)
            sv = st_v[...]
            r0 = b - boff
            start_c = jnp.sum(jnp.where(lane == r0, sv, 0))
            end_c = jnp.sum(jnp.where(lane == r0 + 1, sv, 0))
            return start_c, end_c

        def sweep(mode):
            for core in range(NC):
                start_c, end_c = seg(core)
                astart = (start_c // 8) * 8
                nch = (end_c - astart + CH - 1) // CH
                gbase = core * HALF + astart

                def ch_body(ch, _):
                    pltpu.sync_copy(
                        sorted_hbm.at[pl.ds(gbase + ch * CH, CH)], ent_v)
                    if mode == 1:
                        def id_body(t, _):
                            ev = ent_v[pl.ds(t * 16, 16)]
                            idv = lax.shift_right_logical(ev, 10)
                            idv = jnp.minimum(jnp.maximum(idv, 0), N - 1)
                            ids_v[pl.ds(t * 16, 16)] = idv
                            return 0
                        lax.fori_loop(0, CH // 16, id_body, 0)
                        pltpu.async_copy(enc_hbm.at[ids_v], rows_v, sem).wait()

                    def grp_body(t, _):
                        ev = ent_v[pl.ds(t * 16, 16)]
                        sloc = jnp.bitwise_and(ev, 1023)
                        pos0 = astart + ch * CH + t * 16
                        for l in range(16):
                            valid = jnp.logical_and(pos0 + l >= start_c,
                                                    pos0 + l < end_c)

                            @pl.when(valid)
                            def _():
                                sl = jnp.sum(jnp.where(lane == l, sloc, 0))
                                if mode == 1:
                                    for k4 in range(4):
                                        idxv = lane9 + (k4 * 14400 + sl)
                                        ev4 = rows_v[t * 16 + l,
                                                     pl.ds(k4 * 16, 16)]
                                        cur = plsc.load_gather(tile, [idxv])
                                        plsc.store_scatter(
                                            tile, [idxv],
                                            jnp.maximum(cur, ev4))
                                else:
                                    fill = ninf16 if mode == 0 else zero16f
                                    for k4 in range(4):
                                        idxv = lane9 + (k4 * 14400 + sl)
                                        plsc.store_scatter(tile, [idxv], fill)
                        return 0
                    lax.fori_loop(0, CH // 16, grp_body, 0)
                    return 0

                lax.fori_loop(0, nch, ch_body, 0)

        sweep(0)
        sweep(1)
        pltpu.sync_copy(tile, out_hbm.at[pl.ds(b * TILE, TILE)])
        sweep(2)
        return 0

    lax.fori_loop(0, BPW, bucket_body, 0)


def kernel(batch_size, agent_encodings, encode_coordinates):
    del batch_size
    sorted_packed, starts = _bin_kernel(encode_coordinates)
    out2d = _acc_kernel(agent_encodings, sorted_packed, starts)
    return out2d.reshape(NBATCH, C, 30, 30)
